# baseline (speedup 1.0000x reference)
import jax
import jax.numpy as jnp
from jax.experimental import pallas as pl
from jax.experimental.pallas import tpu as pltpu

_TOPK = 32
_DBLK = 256
_CHUNK = 256
_HEAD = 2


def _proj_kernel(x_ref, w_ref, o_ref):
    o_ref[...] = jnp.dot(x_ref[...], w_ref[...],
                         preferred_element_type=jnp.float32)


def _project(x, w, blk):
    n, k = x.shape
    u = w.shape[1]
    return pl.pallas_call(
        _proj_kernel,
        grid=(n // blk,),
        in_specs=[
            pl.BlockSpec((blk, k), lambda i: (i, 0)),
            pl.BlockSpec((k, u), lambda i: (0, 0)),
        ],
        out_specs=pl.BlockSpec((blk, u), lambda i: (i, 0)),
        out_shape=jax.ShapeDtypeStruct((n, u), jnp.float32),
        compiler_params=pltpu.CompilerParams(
            dimension_semantics=("parallel",)),
    )(x, w)


def _simtopk_kernel(hd_ref, ht_ref, bmat_ref, tri_ref, upre_ref,
                    res_ref, flt_ref, t_ref, need_ref, tot_ref):
    logits = jax.lax.dot_general(
        hd_ref[...], ht_ref[...], (((1,), (1,)), ((), ())),
        preferred_element_type=jnp.float32)
    s = jax.nn.sigmoid(logits)
    res_ref[...] = s
    d, t_num = s.shape
    nc = t_num // _CHUNK

    hi = jnp.max(s, axis=1, keepdims=True)
    eqhi = (s == hi).astype(jnp.bfloat16)
    tot = jax.lax.dot_general(
        eqhi, bmat_ref[...], (((1,), (0,)), ((), ())),
        preferred_element_type=jnp.float32)
    tot_ref[...] = tot
    cnt_hi = jnp.sum(tot, axis=1, keepdims=True)
    fast = jnp.all(cnt_hi >= _TOPK)

    @pl.when(fast)
    def _():
        t_ref[...] = hi
        need_ref[...] = jnp.full((d, 1), float(_TOPK), jnp.float32)

    @pl.when(jnp.logical_not(fast))
    def _():
        key = jax.lax.bitcast_convert_type(s, jnp.int32)
        hik = jax.lax.bitcast_convert_type(hi, jnp.int32)
        lok = jnp.zeros_like(hik)

        def body(_, carry):
            lo, h = carry
            mid = (lo + h + 1) >> 1
            cnt = jnp.sum((key >= mid).astype(jnp.int32), axis=1,
                          keepdims=True)
            ok = cnt >= _TOPK
            return jnp.where(ok, mid, lo), jnp.where(ok, h, mid - 1)

        lok, _hik = jax.lax.fori_loop(0, 31, body, (lok, hik))
        t = jax.lax.bitcast_convert_type(lok, jnp.float32)
        t_ref[...] = t
        eqb = (s == t).astype(jnp.bfloat16)
        tot_ref[...] = jax.lax.dot_general(
            eqb, bmat_ref[...], (((1,), (0,)), ((), ())),
            preferred_element_type=jnp.float32)
        gt_cnt = jnp.sum((s > t).astype(jnp.float32), axis=1, keepdims=True)
        need_ref[...] = _TOPK - gt_cnt

    t = t_ref[...]
    need = need_ref[...]
    pre = jax.lax.dot_general(
        tot_ref[...].astype(jnp.bfloat16), upre_ref[...],
        (((1,), (0,)), ((), ())), preferred_element_type=jnp.float32)
    needc_all = need - pre

    tri = tri_ref[...]

    def chunk_mask(c):
        needc = needc_all[:, c:c + 1]
        sl = s[:, c * _CHUNK:(c + 1) * _CHUNK]
        eqc = (sl == t)
        pref = jax.lax.dot_general(
            eqc.astype(jnp.bfloat16), tri, (((1,), (0,)), ((), ())),
            preferred_element_type=jnp.float32)
        keep = (sl > t) | (eqc & (pref <= needc))
        flt_ref[:, c * _CHUNK:(c + 1) * _CHUNK] = jnp.where(
            keep, sl, jnp.float32(0.0))

    head = min(_HEAD, nc)
    for c in range(head):
        chunk_mask(c)
    if head == nc:
        return

    tail_zero = fast & (jnp.max(needc_all[:, head:head + 1]) < 1.0)

    @pl.when(tail_zero)
    def _():
        flt_ref[:, head * _CHUNK:] = jnp.zeros(
            (d, t_num - head * _CHUNK), jnp.float32)

    @pl.when(jnp.logical_not(tail_zero))
    def _():
        for c in range(head, nc):
            chunk_mask(c)


def kernel(H_d, H_t, W1, W2):
    d_num = H_d.shape[0]
    t_num = H_t.shape[0]
    Hd = _project(H_d, W1, min(1024, d_num))
    Ht = _project(H_t, W2, min(1024, t_num))
    units = Hd.shape[1]
    nc = t_num // _CHUNK
    col = jnp.arange(t_num, dtype=jnp.int32)
    bmat = (col[:, None] // _CHUNK
            == jnp.arange(nc, dtype=jnp.int32)[None, :]).astype(jnp.bfloat16)
    r256 = jnp.arange(_CHUNK, dtype=jnp.int32)
    tri = (r256[:, None] <= r256[None, :]).astype(jnp.bfloat16)
    rnc = jnp.arange(nc, dtype=jnp.int32)
    upre = (rnc[:, None] < rnc[None, :]).astype(jnp.bfloat16)

    res, flt = pl.pallas_call(
        _simtopk_kernel,
        grid=(d_num // _DBLK,),
        in_specs=[
            pl.BlockSpec((_DBLK, units), lambda i: (i, 0)),
            pl.BlockSpec((t_num, units), lambda i: (0, 0)),
            pl.BlockSpec((t_num, nc), lambda i: (0, 0)),
            pl.BlockSpec((_CHUNK, _CHUNK), lambda i: (0, 0)),
            pl.BlockSpec((nc, nc), lambda i: (0, 0)),
        ],
        out_specs=[
            pl.BlockSpec((_DBLK, t_num), lambda i: (i, 0)),
            pl.BlockSpec((_DBLK, t_num), lambda i: (i, 0)),
        ],
        out_shape=[
            jax.ShapeDtypeStruct((d_num, t_num), jnp.float32),
            jax.ShapeDtypeStruct((d_num, t_num), jnp.float32),
        ],
        scratch_shapes=[
            pltpu.VMEM((_DBLK, 1), jnp.float32),
            pltpu.VMEM((_DBLK, 1), jnp.float32),
            pltpu.VMEM((_DBLK, nc), jnp.float32),
        ],
        compiler_params=pltpu.CompilerParams(
            dimension_semantics=("parallel",)),
    )(Hd, Ht, bmat, tri, upre)
    return res, flt


# speedup vs baseline: 1.1338x; 1.1338x over previous
import jax
import jax.numpy as jnp
from jax.experimental import pallas as pl
from jax.experimental.pallas import tpu as pltpu

_TOPK = 32
_DBLK = 256
_CHUNK = 256
_HEAD = 4


def _simtopk_kernel(hd_ref, ht_ref, w1_ref, w2_ref, bmat_ref, tri_ref,
                    upre_ref, res_ref, flt_ref, t_ref, need_ref, tot_ref,
                    htp_ref):
    @pl.when(pl.program_id(0) == 0)
    def _():
        htp_ref[...] = jnp.dot(ht_ref[...], w2_ref[...],
                               preferred_element_type=jnp.float32)

    hd = jnp.dot(hd_ref[...], w1_ref[...],
                 preferred_element_type=jnp.float32)
    logits = jax.lax.dot_general(
        hd, htp_ref[...], (((1,), (1,)), ((), ())),
        preferred_element_type=jnp.float32)
    s = jax.nn.sigmoid(logits)
    res_ref[...] = s
    d, t_num = s.shape
    nc = t_num // _CHUNK

    hi = jnp.max(s, axis=1, keepdims=True)
    eqhi = (s == hi).astype(jnp.bfloat16)
    tot = jax.lax.dot_general(
        eqhi, bmat_ref[...], (((1,), (0,)), ((), ())),
        preferred_element_type=jnp.float32)
    tot_ref[...] = tot
    cnt_hi = jnp.sum(tot, axis=1, keepdims=True)
    fast = jnp.all(cnt_hi >= _TOPK)

    @pl.when(fast)
    def _():
        t_ref[...] = hi
        need_ref[...] = jnp.full((d, 1), float(_TOPK), jnp.float32)

    @pl.when(jnp.logical_not(fast))
    def _():
        key = jax.lax.bitcast_convert_type(s, jnp.int32)
        hik = jax.lax.bitcast_convert_type(hi, jnp.int32)
        lok = jnp.zeros_like(hik)

        def body(_, carry):
            lo, h = carry
            mid = (lo + h + 1) >> 1
            cnt = jnp.sum((key >= mid).astype(jnp.int32), axis=1,
                          keepdims=True)
            ok = cnt >= _TOPK
            return jnp.where(ok, mid, lo), jnp.where(ok, h, mid - 1)

        lok, _hik = jax.lax.fori_loop(0, 31, body, (lok, hik))
        t = jax.lax.bitcast_convert_type(lok, jnp.float32)
        t_ref[...] = t
        eqb = (s == t).astype(jnp.bfloat16)
        tot_ref[...] = jax.lax.dot_general(
            eqb, bmat_ref[...], (((1,), (0,)), ((), ())),
            preferred_element_type=jnp.float32)
        gt_cnt = jnp.sum((s > t).astype(jnp.float32), axis=1, keepdims=True)
        need_ref[...] = _TOPK - gt_cnt

    t = t_ref[...]
    need = need_ref[...]
    pre = jax.lax.dot_general(
        tot_ref[...].astype(jnp.bfloat16), upre_ref[...],
        (((1,), (0,)), ((), ())), preferred_element_type=jnp.float32)
    needc_all = need - pre

    tri = tri_ref[...]

    def chunk_mask(c):
        needc = needc_all[:, c:c + 1]
        sl = s[:, c * _CHUNK:(c + 1) * _CHUNK]
        eqc = (sl == t)
        pref = jax.lax.dot_general(
            eqc.astype(jnp.bfloat16), tri, (((1,), (0,)), ((), ())),
            preferred_element_type=jnp.float32)
        keep = (sl > t) | (eqc & (pref <= needc))
        flt_ref[:, c * _CHUNK:(c + 1) * _CHUNK] = jnp.where(
            keep, sl, jnp.float32(0.0))

    head = min(_HEAD, nc)
    for c in range(head):
        chunk_mask(c)
    if head == nc:
        return

    tail_zero = fast & (jnp.max(needc_all[:, head:head + 1]) < 1.0)

    @pl.when(tail_zero)
    def _():
        flt_ref[:, head * _CHUNK:] = jnp.zeros(
            (d, t_num - head * _CHUNK), jnp.float32)

    @pl.when(jnp.logical_not(tail_zero))
    def _():
        for c in range(head, nc):
            chunk_mask(c)


def kernel(H_d, H_t, W1, W2):
    d_num, d_dim = H_d.shape
    t_num, t_dim = H_t.shape
    units = W1.shape[1]
    nc = t_num // _CHUNK
    col = jnp.arange(t_num, dtype=jnp.int32)
    bmat = (col[:, None] // _CHUNK
            == jnp.arange(nc, dtype=jnp.int32)[None, :]).astype(jnp.bfloat16)
    r256 = jnp.arange(_CHUNK, dtype=jnp.int32)
    tri = (r256[:, None] <= r256[None, :]).astype(jnp.bfloat16)
    rnc = jnp.arange(nc, dtype=jnp.int32)
    upre = (rnc[:, None] < rnc[None, :]).astype(jnp.bfloat16)

    res, flt = pl.pallas_call(
        _simtopk_kernel,
        grid=(d_num // _DBLK,),
        in_specs=[
            pl.BlockSpec((_DBLK, d_dim), lambda i: (i, 0)),
            pl.BlockSpec((t_num, t_dim), lambda i: (0, 0)),
            pl.BlockSpec((d_dim, units), lambda i: (0, 0)),
            pl.BlockSpec((t_dim, units), lambda i: (0, 0)),
            pl.BlockSpec((t_num, nc), lambda i: (0, 0)),
            pl.BlockSpec((_CHUNK, _CHUNK), lambda i: (0, 0)),
            pl.BlockSpec((nc, nc), lambda i: (0, 0)),
        ],
        out_specs=[
            pl.BlockSpec((_DBLK, t_num), lambda i: (i, 0)),
            pl.BlockSpec((_DBLK, t_num), lambda i: (i, 0)),
        ],
        out_shape=[
            jax.ShapeDtypeStruct((d_num, t_num), jnp.float32),
            jax.ShapeDtypeStruct((d_num, t_num), jnp.float32),
        ],
        scratch_shapes=[
            pltpu.VMEM((_DBLK, 1), jnp.float32),
            pltpu.VMEM((_DBLK, 1), jnp.float32),
            pltpu.VMEM((_DBLK, nc), jnp.float32),
            pltpu.VMEM((t_num, units), jnp.float32),
        ],
        compiler_params=pltpu.CompilerParams(
            dimension_semantics=("arbitrary",)),
    )(H_d, H_t, W1, W2, bmat, tri, upre)
    return res, flt
